# Initial kernel scaffold; baseline (speedup 1.0000x reference)
import jax
import jax.numpy as jnp
from jax.experimental import pallas as pl


def kernel(x, wave_w, wave_b, query_w, query_b, key_weight, out_w, se1_w, se1_b, se2_w, se2_b):
    raise NotImplementedError("write your pallas kernel here")


# speedup vs baseline: 8.0385x; 8.0385x over previous
import functools

import jax
import jax.numpy as jnp
from jax.experimental import pallas as pl

C = 1024
H = 16
D = C // H
POS_DIM = 16
MAX_SAMPLES = 32
HALF_S = MAX_SAMPLES // 2
S = 2 * HALF_S + 1
MAX_FREQ = 16.0
MIN_FREQ = 1.0
SCALE = POS_DIM ** -0.5
L = 2048

PRE_BL = 256
G_BL = 128
G_W = 768
POST_BL = 256


def _silu(v):
    return v * jax.nn.sigmoid(v)


def _pre_kernel(x_ref, wave_wT_ref, wave_b_ref, query_wT_ref, query_b_ref,
                kw_mat_ref, attn_ref, idx_ref):
    i = pl.program_id(0)
    xb = x_ref[0]
    wave = _silu(jnp.dot(xb, wave_wT_ref[...], preferred_element_type=jnp.float32)
                 + wave_b_ref[...])
    queries = _silu(jnp.dot(xb, query_wT_ref[...], preferred_element_type=jnp.float32)
                    + query_b_ref[...])
    freq = jax.nn.sigmoid(wave[:, 0:H]) * (MAX_FREQ - MIN_FREQ) + MIN_FREQ
    phase = jnp.tanh(wave[:, H:2 * H]) * MAX_FREQ
    decay = jax.nn.sigmoid(wave[:, 2 * H:3 * H]) * 9.5 + 0.5
    freq_avg = jnp.mean(freq, axis=1, keepdims=True)
    phase_avg = jnp.mean(phase, axis=1, keepdims=True)
    decay_avg = jnp.mean(decay, axis=1, keepdims=True)
    qk = jnp.dot(queries, kw_mat_ref[...], preferred_element_type=jnp.float32)

    stride = (jax.lax.broadcasted_iota(jnp.int32, (1, S), 1)
              - HALF_S).astype(jnp.float32)
    centers = (jax.lax.broadcasted_iota(jnp.int32, (PRE_BL, 1), 0)
               + i * PRE_BL).astype(jnp.float32)
    pos = centers + stride * freq_avg + phase_avg
    valid = (pos >= 0.0) & (pos < float(L))
    validf = valid.astype(jnp.float32)
    idx = jnp.clip(pos.astype(jnp.int32), 0, L - 1)
    rel = jnp.abs(stride) * freq_avg
    denv = jnp.exp(-rel / jnp.maximum(decay_avg, 0.1)) * validf
    relS = rel * SCALE

    for h in range(H):
        sc = qk[:, h:h + 1] * relS
        sc = jnp.where(valid, sc, -1e30)
        m = jnp.max(sc, axis=1, keepdims=True)
        e = jnp.exp(sc - m)
        a = e / jnp.sum(e, axis=1, keepdims=True)
        a = a * denv
        a = a / (jnp.sum(a, axis=1, keepdims=True) + 1e-8)
        attn_ref[:, h, :] = a
    idx_ref[...] = idx


def _gather_kernel(x_ref, attn_ref, idx_ref, out_ref):
    i = pl.program_id(0)
    l0 = i * G_BL
    w0 = pl.multiple_of(jnp.clip(l0 - 272, 0, L - G_W), 8)
    xw = x_ref[pl.ds(w0, G_W), :]
    lane = jax.lax.broadcasted_iota(jnp.int32, (G_BL, G_W), 1)
    hsel = jax.lax.broadcasted_iota(jnp.int32, (H, C), 1) // D
    hrow = jax.lax.broadcasted_iota(jnp.int32, (H, C), 0)
    expand = (hsel == hrow).astype(jnp.float32)
    acc = jnp.zeros((G_BL, C), jnp.float32)
    for s in range(S):
        rel_idx = idx_ref[:, s:s + 1] - w0
        p = (rel_idx == lane).astype(jnp.float32)
        ws = jnp.dot(attn_ref[:, s, :], expand,
                     preferred_element_type=jnp.float32)
        acc = acc + jnp.dot(p, xw, preferred_element_type=jnp.float32) * ws
    out_ref[...] = acc


def _post_kernel(o_ref, se1_wT_ref, se1_b_ref, se2_wT_ref, se2_b_ref,
                 out_wT_ref, out_ref):
    o = o_ref[...]
    h1 = _silu(jnp.dot(o, se1_wT_ref[...], preferred_element_type=jnp.float32)
               + se1_b_ref[...])
    se = jax.nn.sigmoid(jnp.dot(h1, se2_wT_ref[...], preferred_element_type=jnp.float32)
                        + se2_b_ref[...])
    g = o * se
    out_ref[0] = _silu(jnp.dot(g, out_wT_ref[...], preferred_element_type=jnp.float32))


@jax.jit
def kernel(x, wave_w, wave_b, query_w, query_b, key_weight, out_w, se1_w,
           se1_b, se2_w, se2_b):
    B = x.shape[0]
    x2 = x.reshape(L, C)
    kw_mat = jnp.zeros((H * POS_DIM, H), jnp.float32)
    kw_mat = kw_mat.at[jnp.arange(H * POS_DIM),
                       jnp.arange(H * POS_DIM) // POS_DIM].set(
        jnp.tile(key_weight, H))

    n_pre = L // PRE_BL
    attn, idx = pl.pallas_call(
        _pre_kernel,
        grid=(n_pre,),
        in_specs=[
            pl.BlockSpec((1, PRE_BL, C), lambda i: (0, i, 0)),
            pl.BlockSpec((C, 3 * H), lambda i: (0, 0)),
            pl.BlockSpec((1, 3 * H), lambda i: (0, 0)),
            pl.BlockSpec((C, H * POS_DIM), lambda i: (0, 0)),
            pl.BlockSpec((1, H * POS_DIM), lambda i: (0, 0)),
            pl.BlockSpec((H * POS_DIM, H), lambda i: (0, 0)),
        ],
        out_specs=[
            pl.BlockSpec((PRE_BL, H, S), lambda i: (i, 0, 0)),
            pl.BlockSpec((PRE_BL, S), lambda i: (i, 0)),
        ],
        out_shape=[
            jax.ShapeDtypeStruct((L, H, S), jnp.float32),
            jax.ShapeDtypeStruct((L, S), jnp.int32),
        ],
    )(x, wave_w.T, wave_b[None], query_w.T, query_b[None], kw_mat)
    attn = attn.transpose(0, 2, 1)

    n_g = L // G_BL
    out1 = pl.pallas_call(
        _gather_kernel,
        grid=(n_g,),
        in_specs=[
            pl.BlockSpec((L, C), lambda i: (0, 0)),
            pl.BlockSpec((G_BL, S, H), lambda i: (i, 0, 0)),
            pl.BlockSpec((G_BL, S), lambda i: (i, 0)),
        ],
        out_specs=pl.BlockSpec((G_BL, C), lambda i: (i, 0)),
        out_shape=jax.ShapeDtypeStruct((L, C), jnp.float32),
    )(x2, attn, idx)

    n_post = L // POST_BL
    out = pl.pallas_call(
        _post_kernel,
        grid=(n_post,),
        in_specs=[
            pl.BlockSpec((POST_BL, C), lambda i: (i, 0)),
            pl.BlockSpec((C, C // 4), lambda i: (0, 0)),
            pl.BlockSpec((1, C // 4), lambda i: (0, 0)),
            pl.BlockSpec((C // 4, C), lambda i: (0, 0)),
            pl.BlockSpec((1, C), lambda i: (0, 0)),
            pl.BlockSpec((C, C), lambda i: (0, 0)),
        ],
        out_specs=pl.BlockSpec((1, POST_BL, C), lambda i: (0, i, 0)),
        out_shape=jax.ShapeDtypeStruct((B, L, C), jnp.float32),
    )(out1, se1_w.T, se1_b[None], se2_w.T, se2_b[None], out_w.T)
    return out
